# baseline (speedup 1.0000x reference)
import functools
import jax
import jax.numpy as jnp
from jax import lax
from jax.experimental import pallas as pl
from jax.experimental.pallas import tpu as pltpu, tpu_sc as plsc

_CH = 8192


def kernel(sample):
    B, C, T = sample.shape
    NC, NS = 2, 16
    NW = NC * NS
    cols_per_w = T // NW
    n_sub = cols_per_w // _CH
    n_it = B * n_sub
    x = sample.reshape(B * C, T)

    mesh = plsc.VectorSubcoreMesh(
        core_axis_name="c", subcore_axis_name="s", num_cores=NC, num_subcores=NS
    )

    @functools.partial(
        pl.kernel,
        out_type=jax.ShapeDtypeStruct((B * 2, T), jnp.float32),
        mesh=mesh,
        scratch_types=[
            pltpu.VMEM((3, _CH), jnp.float32),
            pltpu.VMEM((1, _CH), jnp.float32),
            pltpu.VMEM_SHARED((NS * 6, _CH), jnp.float32),
        ],
    )
    def mix(x_hbm, out_hbm, tbuf_in, tbuf_o, spm):
        sid = lax.axis_index("s")
        wid = sid * NC + lax.axis_index("c")
        col0 = wid * cols_per_w
        base = sid * 6

        def body(it, _):
            b = it // n_sub
            off = pl.multiple_of(col0 + (it % n_sub) * _CH, _CH)
            sl = pl.ds(off, _CH)
            pltpu.sync_copy(x_hbm.at[pl.ds(4 * b + 0, 1), sl], spm.at[pl.ds(base + 0, 1)])
            pltpu.sync_copy(x_hbm.at[pl.ds(4 * b + 1, 1), sl], spm.at[pl.ds(base + 1, 1)])
            pltpu.sync_copy(x_hbm.at[pl.ds(4 * b + 2, 1), sl], spm.at[pl.ds(base + 2, 1)])
            pltpu.sync_copy(spm.at[pl.ds(base + 0, 1)], tbuf_in.at[pl.ds(0, 1)])
            pltpu.sync_copy(spm.at[pl.ds(base + 1, 1)], tbuf_in.at[pl.ds(1, 1)])
            pltpu.sync_copy(spm.at[pl.ds(base + 2, 1)], tbuf_in.at[pl.ds(2, 1)])

            @plsc.parallel_loop(0, _CH, 16, unroll=8)
            def compute(i):
                tbuf_o[0, pl.ds(i, 16)] = (
                    tbuf_in[0, pl.ds(i, 16)]
                    + tbuf_in[1, pl.ds(i, 16)]
                    + tbuf_in[2, pl.ds(i, 16)]
                )

            pltpu.sync_copy(tbuf_o.at[pl.ds(0, 1)], spm.at[pl.ds(base + 3, 1)])
            pltpu.sync_copy(spm.at[pl.ds(base + 3, 1)], out_hbm.at[pl.ds(2 * b, 1), sl])
            pltpu.sync_copy(x_hbm.at[pl.ds(4 * b + 3, 1), sl], spm.at[pl.ds(base + 4, 1)])
            pltpu.sync_copy(spm.at[pl.ds(base + 4, 1)], out_hbm.at[pl.ds(2 * b + 1, 1), sl])
            return 0

        lax.fori_loop(0, n_it, body, 0)

    out = mix(x)
    return out.reshape(B, 2, T)


# speedup vs baseline: 6.4236x; 6.4236x over previous
import jax
import jax.numpy as jnp
from jax.experimental import pallas as pl


_BT = 131072


def _mix_body(s_ref, o_ref):
    s = s_ref[...]
    o_ref[:, 0, :] = s[:, 0, :] + s[:, 1, :] + s[:, 2, :]
    o_ref[:, 1, :] = s[:, 3, :]


def kernel(sample):
    B, C, T = sample.shape
    grid = (B, T // _BT)
    return pl.pallas_call(
        _mix_body,
        grid=grid,
        in_specs=[pl.BlockSpec((1, C, _BT), lambda b, i: (b, 0, i))],
        out_specs=pl.BlockSpec((1, 2, _BT), lambda b, i: (b, 0, i)),
        out_shape=jax.ShapeDtypeStruct((B, 2, T), sample.dtype),
    )(sample)


# speedup vs baseline: 8.5814x; 1.3359x over previous
import jax
import jax.numpy as jnp
from jax.experimental import pallas as pl


_BT = 131072


def _mix_body(s_ref, o_ref):
    s = s_ref[...]
    o_ref[:, 0, :] = s[:, 0, :] + s[:, 1, :] + s[:, 2, :]
    o_ref[:, 1, :] = s[:, 3, :]


def kernel(sample):
    B, C, T = sample.shape
    grid = (T // _BT,)
    return pl.pallas_call(
        _mix_body,
        grid=grid,
        in_specs=[pl.BlockSpec((B, C, _BT), lambda i: (0, 0, i))],
        out_specs=pl.BlockSpec((B, 2, _BT), lambda i: (0, 0, i)),
        out_shape=jax.ShapeDtypeStruct((B, 2, T), sample.dtype),
    )(sample)


# speedup vs baseline: 8.9803x; 1.0465x over previous
import jax
import jax.numpy as jnp
from jax.experimental import pallas as pl
from jax.experimental.pallas import tpu as pltpu

_BT = 32768
_DEPTH = 3


def kernel(sample):
    B, C, T = sample.shape
    nch = T // _BT

    def body(x_hbm, o_hbm, in_buf, out_buf, sem_in, sem_out):
        def in_copy(k, slot):
            return pltpu.make_async_copy(
                x_hbm.at[:, :, pl.ds(k * _BT, _BT)], in_buf.at[slot], sem_in.at[slot]
            )

        def out_copy(k, slot):
            return pltpu.make_async_copy(
                out_buf.at[slot], o_hbm.at[:, :, pl.ds(k * _BT, _BT)], sem_out.at[slot]
            )

        for k in range(min(_DEPTH, nch)):
            in_copy(k, k).start()

        for k in range(nch):
            slot = k % _DEPTH
            in_copy(k, slot).wait()
            if k >= _DEPTH:
                out_copy(k - _DEPTH, slot).wait()
            s = in_buf[slot]
            out_buf[slot, :, 0, :] = s[:, 0, :] + s[:, 1, :] + s[:, 2, :]
            out_buf[slot, :, 1, :] = s[:, 3, :]
            out_copy(k, slot).start()
            if k + _DEPTH < nch:
                in_copy(k + _DEPTH, slot).start()

        for k in range(max(nch - _DEPTH, 0), nch):
            out_copy(k, k % _DEPTH).wait()

    return pl.pallas_call(
        body,
        in_specs=[pl.BlockSpec(memory_space=pltpu.HBM)],
        out_specs=pl.BlockSpec(memory_space=pltpu.HBM),
        out_shape=jax.ShapeDtypeStruct((B, 2, T), sample.dtype),
        scratch_shapes=[
            pltpu.VMEM((_DEPTH, B, C, _BT), jnp.float32),
            pltpu.VMEM((_DEPTH, B, 2, _BT), jnp.float32),
            pltpu.SemaphoreType.DMA((_DEPTH,)),
            pltpu.SemaphoreType.DMA((_DEPTH,)),
        ],
    )(sample)
